# baseline (speedup 1.0000x reference)
import functools

import jax
import jax.numpy as jnp
from jax import lax
from jax.experimental import pallas as pl
from jax.experimental.pallas import tpu as pltpu
from jax.experimental.pallas import tpu_sc as plsc

N = 10000
NP = 10240
E = 320000
EP = 327680
D = 128
H = 256
Z = 64
C = 64
NCH = EP // (16 * C)
RPT = NP // 16
NQ = 8
QCH = NCH // NQ

_MESH = plsc.VectorSubcoreMesh(
    core_axis_name="c", subcore_axis_name="s", num_cores=2, num_subcores=16)


def _fill2d(ref, rows, cols, val):
    def row(r, carry):
        for k in range(cols // 16):
            ref[r, pl.ds(k * 16, 16)] = jnp.full((16,), val, jnp.float32)
        return carry
    lax.fori_loop(0, rows, row, 0)


def _fill1d(ref, n, val):
    def blk(k, carry):
        ref[pl.ds(k * 16, 16)] = jnp.full((16,), val, jnp.float32)
        return carry
    lax.fori_loop(0, n // 16, blk, 0)


def _seg_loop(s, x_hbm, src_hbm, dst_hbm, src_v, dst_v, g0, g1, acc,
              sem0, sem1, cacc=None, ones_v=None):
    for q in range(NQ):
        base = s * NCH + q * QCH
        pltpu.sync_copy(src_hbm.at[pl.ds(base, QCH)], src_v)
        pltpu.sync_copy(dst_hbm.at[pl.ds(base, QCH)], dst_v)
        pltpu.async_copy(x_hbm.at[src_v.at[0]], g0, sem0)

        def body(i, carry):
            j = 2 * i
            pltpu.async_copy(x_hbm.at[src_v.at[j + 1]], g1, sem1)
            pltpu.make_async_copy(x_hbm.at[src_v.at[j]], g0, sem0).wait()
            pltpu.sync_copy(g0, acc.at[dst_v.at[j]], add=True)
            if cacc is not None:
                pltpu.sync_copy(ones_v, cacc.at[dst_v.at[j]], add=True)

            @pl.when(j + 2 < QCH)
            def _():
                pltpu.async_copy(x_hbm.at[src_v.at[j + 2]], g0, sem0)

            pltpu.make_async_copy(x_hbm.at[src_v.at[j + 1]], g1, sem1).wait()
            pltpu.sync_copy(g1, acc.at[dst_v.at[j + 1]], add=True)
            if cacc is not None:
                pltpu.sync_copy(ones_v, cacc.at[dst_v.at[j + 1]], add=True)
            return carry

        lax.fori_loop(0, QCH // 2, body, 0)


def _seg_loop4(s, x_hbm, src_hbm, dst_hbm, src_v, dst_v, gs, acc, sems):
    for q in range(NQ):
        base = s * NCH + q * QCH
        pltpu.sync_copy(src_hbm.at[pl.ds(base, QCH)], src_v)
        pltpu.sync_copy(dst_hbm.at[pl.ds(base, QCH)], dst_v)
        for b in range(4):
            pltpu.async_copy(x_hbm.at[src_v.at[b]], gs[b], sems[b])

        def body(i, carry):
            j = 4 * i
            for b in range(4):
                pltpu.make_async_copy(
                    x_hbm.at[src_v.at[j + b]], gs[b], sems[b]).wait()
                pltpu.sync_copy(gs[b], acc.at[dst_v.at[j + b]], add=True)

                @pl.when(j + b + 4 < QCH)
                def _():
                    pltpu.async_copy(
                        x_hbm.at[src_v.at[j + b + 4]], gs[b], sems[b])
            return carry

        lax.fori_loop(0, QCH // 4, body, 0)


def _agg1_body(xa, xb, srcA, dstA, srcB, dstB,
               out_a, out_b,
               acc, src_v, dst_v, g0, g1, g2, g3,
               sem0, sem1, sem2, sem3):
    c = lax.axis_index("c")
    s = lax.axis_index("s")
    _fill2d(g0, C, D, 0.0)
    for k in range(RPT // C):
        pltpu.sync_copy(g0, acc.at[pl.ds(s * RPT + k * C, C)])
    plsc.subcore_barrier()
    gs = [g0, g1, g2, g3]
    sems = [sem0, sem1, sem2, sem3]

    def run(x_hbm, src_hbm, dst_hbm, out_hbm):
        _seg_loop4(s, x_hbm, src_hbm, dst_hbm, src_v, dst_v, gs, acc, sems)
        plsc.subcore_barrier()
        pltpu.sync_copy(acc.at[pl.ds(s * RPT, RPT)],
                        out_hbm.at[pl.ds(s * RPT, RPT)])

    @pl.when(c == 0)
    def _():
        run(xa, srcA, dstA, out_a)

    @pl.when(c == 1)
    def _():
        run(xb, srcB, dstB, out_b)


_agg1 = functools.partial(
    pl.kernel,
    out_type=[
        jax.ShapeDtypeStruct((NP, D), jnp.float32),
        jax.ShapeDtypeStruct((NP, D), jnp.float32),
    ],
    mesh=_MESH,
    scratch_types=[
        pltpu.VMEM_SHARED((NP, D), jnp.float32),
        pltpu.VMEM((QCH, C), jnp.int32),
        pltpu.VMEM((QCH, C), jnp.int32),
        pltpu.VMEM((C, D), jnp.float32),
        pltpu.VMEM((C, D), jnp.float32),
        pltpu.VMEM((C, D), jnp.float32),
        pltpu.VMEM((C, D), jnp.float32),
        pltpu.SemaphoreType.DMA,
        pltpu.SemaphoreType.DMA,
        pltpu.SemaphoreType.DMA,
        pltpu.SemaphoreType.DMA,
    ],
)(_agg1_body)


def _cnt_body(dstA, dstB, cnt_a, cnt_b,
              cacc, dst_v, zcnt, ones_v):
    c = lax.axis_index("c")
    s = lax.axis_index("s")
    _fill1d(zcnt, RPT, 0.0)
    _fill1d(ones_v, C, 1.0)
    pltpu.sync_copy(zcnt, cacc.at[pl.ds(s * RPT, RPT)])
    plsc.subcore_barrier()

    def run(dst_hbm, cnt_hbm):
        for q in range(NQ):
            base = s * NCH + q * QCH
            pltpu.sync_copy(dst_hbm.at[pl.ds(base, QCH)], dst_v)

            def body(j, carry):
                pltpu.sync_copy(ones_v, cacc.at[dst_v.at[j]], add=True)
                return carry

            lax.fori_loop(0, QCH, body, 0)
        plsc.subcore_barrier()
        pltpu.sync_copy(cacc.at[pl.ds(s * RPT, RPT)],
                        cnt_hbm.at[pl.ds(s * RPT, RPT)])

    @pl.when(c == 0)
    def _():
        run(dstA, cnt_a)

    @pl.when(c == 1)
    def _():
        run(dstB, cnt_b)


_cnt = functools.partial(
    pl.kernel,
    out_type=[
        jax.ShapeDtypeStruct((NP,), jnp.float32),
        jax.ShapeDtypeStruct((NP,), jnp.float32),
    ],
    mesh=_MESH,
    scratch_types=[
        pltpu.VMEM_SHARED((NP,), jnp.float32),
        pltpu.VMEM((QCH, C), jnp.int32),
        pltpu.VMEM((RPT,), jnp.float32),
        pltpu.VMEM((C,), jnp.float32),
    ],
)(_cnt_body)


def _agg2_body(ya, yb, srcA, dstA, srcB, dstB,
               out_a, out_b,
               acc, src_v, dst_v, g0, g1, g2, g3,
               sem0, sem1, sem2, sem3):
    c = lax.axis_index("c")
    s = lax.axis_index("s")
    _fill2d(g0, C, Z, 0.0)
    for k in range(RPT // C):
        pltpu.sync_copy(g0, acc.at[pl.ds(s * RPT + k * C, C)])
    plsc.subcore_barrier()
    gs = [g0, g1, g2, g3]
    sems = [sem0, sem1, sem2, sem3]

    def run(x_hbm, src_hbm, dst_hbm, out_hbm):
        _seg_loop4(s, x_hbm, src_hbm, dst_hbm, src_v, dst_v, gs, acc, sems)
        plsc.subcore_barrier()
        pltpu.sync_copy(acc.at[pl.ds(s * RPT, RPT)],
                        out_hbm.at[pl.ds(s * RPT, RPT)])

    @pl.when(c == 0)
    def _():
        run(ya, srcA, dstA, out_a)

    @pl.when(c == 1)
    def _():
        run(yb, srcB, dstB, out_b)


_agg2 = functools.partial(
    pl.kernel,
    out_type=[
        jax.ShapeDtypeStruct((NP, Z), jnp.float32),
        jax.ShapeDtypeStruct((NP, Z), jnp.float32),
    ],
    mesh=_MESH,
    compiler_params=pltpu.CompilerParams(use_tc_tiling_on_sc=False),
    scratch_types=[
        pltpu.VMEM_SHARED((NP, Z), jnp.float32),
        pltpu.VMEM((QCH, C), jnp.int32),
        pltpu.VMEM((QCH, C), jnp.int32),
        pltpu.VMEM((C, Z), jnp.float32),
        pltpu.VMEM((C, Z), jnp.float32),
        pltpu.VMEM((C, Z), jnp.float32),
        pltpu.VMEM((C, Z), jnp.float32),
        pltpu.SemaphoreType.DMA,
        pltpu.SemaphoreType.DMA,
        pltpu.SemaphoreType.DMA,
        pltpu.SemaphoreType.DMA,
    ],
)(_agg2_body)


BM = 512
BMF = 400


def _tc1_body(m_l, cnt_l, x_l, wl1A, wr1A, b1A, wsA, waA,
              m_p, cnt_p, x_p, wl1B, wr1B, b1B, wsB, waB,
              ysl_ref, yal_ref, ysp_ref, yap_ref):
    def one(m_ref, cnt_ref, x_ref, wl1, wr1, b1, ws, wa, ys_ref, ya_ref):
        r = 1.0 / jnp.maximum(cnt_ref[...], 1.0)
        m = m_ref[...] * r
        h = jnp.dot(m, wl1[...], preferred_element_type=jnp.float32)
        h += jnp.dot(x_ref[...], wr1[...], preferred_element_type=jnp.float32)
        h += b1[...]
        h = jnp.maximum(h, 0.0)
        ys_ref[...] = jnp.dot(h, ws[...], preferred_element_type=jnp.float32)
        ya_ref[...] = jnp.dot(h, wa[...], preferred_element_type=jnp.float32)
    one(m_l, cnt_l, x_l, wl1A, wr1A, b1A, wsA, waA, ysl_ref, yal_ref)
    one(m_p, cnt_p, x_p, wl1B, wr1B, b1B, wsB, waB, ysp_ref, yap_ref)


def _tc1(m_l, cnt_l, x_l, wA, m_p, cnt_p, x_p, wB):
    grid = (N // BMF,)
    full = lambda a: pl.BlockSpec(a.shape, lambda i: tuple(0 for _ in a.shape))
    rowd = pl.BlockSpec((BMF, D), lambda i: (i, 0))
    rowc = pl.BlockSpec((BMF, 1), lambda i: (i, 0))
    rowz = pl.BlockSpec((BMF, Z), lambda i: (i, 0))
    return pl.pallas_call(
        _tc1_body,
        grid=grid,
        in_specs=[rowd, rowc, rowd] + [full(w) for w in wA]
                 + [rowd, rowc, rowd] + [full(w) for w in wB],
        out_specs=[rowz, rowz, rowz, rowz],
        out_shape=[jax.ShapeDtypeStruct((N, Z), jnp.float32)] * 4,
    )(m_l, cnt_l, x_l, *wA, m_p, cnt_p, x_p, *wB)


def _fin_body(m2l_ref, cntl_ref, ysl_ref, b2l_ref,
              m2p_ref, cntp_ref, ysp_ref, b2p_ref, zl_ref, zp_ref):
    rl = 1.0 / jnp.maximum(cntl_ref[...], 1.0)
    zl_ref[...] = m2l_ref[...] * rl + ysl_ref[...] + b2l_ref[...]
    rp = 1.0 / jnp.maximum(cntp_ref[...], 1.0)
    zp_ref[...] = m2p_ref[...] * rp + ysp_ref[...] + b2p_ref[...]


def _fin(m2l, cntl, ysl, b2l, m2p, cntp, ysp, b2p):
    grid = (N // BMF,)
    rowz = pl.BlockSpec((BMF, Z), lambda i: (i, 0))
    rowc = pl.BlockSpec((BMF, 1), lambda i: (i, 0))
    bias = pl.BlockSpec((1, Z), lambda i: (0, 0))
    return pl.pallas_call(
        _fin_body,
        grid=grid,
        in_specs=[rowz, rowc, rowz, bias, rowz, rowc, rowz, bias],
        out_specs=[rowz, rowz],
        out_shape=[
            jax.ShapeDtypeStruct((N, Z), jnp.float32),
            jax.ShapeDtypeStruct((N, Z), jnp.float32),
        ],
    )(m2l, cntl, ysl, b2l, m2p, cntp, ysp, b2p)


def kernel(x_paper, x_label, edge_index_p2l, edge_index_l2p,
           Wl1_p2l, Wr1_p2l, b1_p2l, Wl1_l2p, Wr1_l2p, b1_l2p,
           Wl2_p2l, Wr2_p2l, b2_p2l, Wl2_l2p, Wr2_l2p, b2_l2p):
    xp = x_paper
    xl = x_label

    def prep(ei):
        src = ei[0].astype(jnp.int32)
        dst = ei[1].astype(jnp.int32)
        pad = EP - E
        src = jnp.concatenate([src, jnp.zeros((pad,), jnp.int32)])
        dst = jnp.concatenate([dst, jnp.full((pad,), N, jnp.int32)])
        return src.reshape(16 * NCH, C), dst.reshape(16 * NCH, C)

    srcA, dstA = prep(edge_index_p2l)
    srcB, dstB = prep(edge_index_l2p)

    cnt_lab, cnt_pap = _cnt(dstA, dstB)
    cl = cnt_lab[:, None]
    cp = cnt_pap[:, None]

    m_lab, m_pap = _agg1(xp, xl, srcA, dstA, srcB, dstB)

    ys_lab, ya_lab, ys_pap, ya_pap = _tc1(
        m_lab, cl, xl, (Wl1_p2l, Wr1_p2l, b1_p2l[None], Wr2_p2l, Wl2_l2p),
        m_pap, cp, xp, (Wl1_l2p, Wr1_l2p, b1_l2p[None], Wr2_l2p, Wl2_p2l))

    m2_lab, m2_pap = _agg2(ya_pap, ya_lab, srcA, dstA, srcB, dstB)

    zl, zp = _fin(m2_lab, cl, ys_lab, b2_p2l[None],
                  m2_pap, cp, ys_pap, b2_l2p[None])
    return (zp, zl)


# speedup vs baseline: 1.0220x; 1.0220x over previous
import functools

import jax
import jax.numpy as jnp
from jax import lax
from jax.experimental import pallas as pl
from jax.experimental.pallas import tpu as pltpu
from jax.experimental.pallas import tpu_sc as plsc

N = 10000
NP = 10240
E = 320000
EP = 327680
D = 128
H = 256
Z = 64
C = 128
NCH = EP // (16 * C)
RPT = NP // 16
NQ = 4
QCH = NCH // NQ

_MESH = plsc.VectorSubcoreMesh(
    core_axis_name="c", subcore_axis_name="s", num_cores=2, num_subcores=16)


def _fill2d(ref, rows, cols, val):
    def row(r, carry):
        for k in range(cols // 16):
            ref[r, pl.ds(k * 16, 16)] = jnp.full((16,), val, jnp.float32)
        return carry
    lax.fori_loop(0, rows, row, 0)


def _fill1d(ref, n, val):
    def blk(k, carry):
        ref[pl.ds(k * 16, 16)] = jnp.full((16,), val, jnp.float32)
        return carry
    lax.fori_loop(0, n // 16, blk, 0)


def _seg_loop(s, x_hbm, src_hbm, dst_hbm, src_v, dst_v, g0, g1, acc,
              sem0, sem1, cacc=None, ones_v=None):
    for q in range(NQ):
        base = s * NCH + q * QCH
        pltpu.sync_copy(src_hbm.at[pl.ds(base, QCH)], src_v)
        pltpu.sync_copy(dst_hbm.at[pl.ds(base, QCH)], dst_v)
        pltpu.async_copy(x_hbm.at[src_v.at[0]], g0, sem0)

        def body(i, carry):
            j = 2 * i
            pltpu.async_copy(x_hbm.at[src_v.at[j + 1]], g1, sem1)
            pltpu.make_async_copy(x_hbm.at[src_v.at[j]], g0, sem0).wait()
            pltpu.sync_copy(g0, acc.at[dst_v.at[j]], add=True)
            if cacc is not None:
                pltpu.sync_copy(ones_v, cacc.at[dst_v.at[j]], add=True)

            @pl.when(j + 2 < QCH)
            def _():
                pltpu.async_copy(x_hbm.at[src_v.at[j + 2]], g0, sem0)

            pltpu.make_async_copy(x_hbm.at[src_v.at[j + 1]], g1, sem1).wait()
            pltpu.sync_copy(g1, acc.at[dst_v.at[j + 1]], add=True)
            if cacc is not None:
                pltpu.sync_copy(ones_v, cacc.at[dst_v.at[j + 1]], add=True)
            return carry

        lax.fori_loop(0, QCH // 2, body, 0)


def _seg_loop4(s, x_hbm, src_hbm, dst_hbm, src_v, dst_v, gs, acc, sems):
    for q in range(NQ):
        base = s * NCH + q * QCH
        pltpu.sync_copy(src_hbm.at[pl.ds(base, QCH)], src_v)
        pltpu.sync_copy(dst_hbm.at[pl.ds(base, QCH)], dst_v)
        for b in range(4):
            pltpu.async_copy(x_hbm.at[src_v.at[b]], gs[b], sems[b])

        def body(i, carry):
            j = 4 * i
            for b in range(4):
                pltpu.make_async_copy(
                    x_hbm.at[src_v.at[j + b]], gs[b], sems[b]).wait()
                pltpu.sync_copy(gs[b], acc.at[dst_v.at[j + b]], add=True)

                @pl.when(j + b + 4 < QCH)
                def _():
                    pltpu.async_copy(
                        x_hbm.at[src_v.at[j + b + 4]], gs[b], sems[b])
            return carry

        lax.fori_loop(0, QCH // 4, body, 0)


def _agg1_body(xa, xb, srcA, dstA, srcB, dstB,
               out_a, out_b,
               acc, src_v, dst_v, g0, g1, sem0, sem1):
    c = lax.axis_index("c")
    s = lax.axis_index("s")
    _fill2d(g0, C, D, 0.0)
    for k in range(RPT // C):
        pltpu.sync_copy(g0, acc.at[pl.ds(s * RPT + k * C, C)])
    plsc.subcore_barrier()

    def run(x_hbm, src_hbm, dst_hbm, out_hbm):
        _seg_loop(s, x_hbm, src_hbm, dst_hbm, src_v, dst_v, g0, g1, acc,
                  sem0, sem1)
        plsc.subcore_barrier()
        pltpu.sync_copy(acc.at[pl.ds(s * RPT, RPT)],
                        out_hbm.at[pl.ds(s * RPT, RPT)])

    @pl.when(c == 0)
    def _():
        run(xa, srcA, dstA, out_a)

    @pl.when(c == 1)
    def _():
        run(xb, srcB, dstB, out_b)


_agg1 = functools.partial(
    pl.kernel,
    out_type=[
        jax.ShapeDtypeStruct((NP, D), jnp.float32),
        jax.ShapeDtypeStruct((NP, D), jnp.float32),
    ],
    mesh=_MESH,
    scratch_types=[
        pltpu.VMEM_SHARED((NP, D), jnp.float32),
        pltpu.VMEM((QCH, C), jnp.int32),
        pltpu.VMEM((QCH, C), jnp.int32),
        pltpu.VMEM((C, D), jnp.float32),
        pltpu.VMEM((C, D), jnp.float32),
        pltpu.SemaphoreType.DMA,
        pltpu.SemaphoreType.DMA,
    ],
)(_agg1_body)


def _cnt_body(dstA, dstB, cnt_a, cnt_b,
              cacc, dst_v, zcnt, ones_v):
    c = lax.axis_index("c")
    s = lax.axis_index("s")
    _fill1d(zcnt, RPT, 0.0)
    _fill1d(ones_v, C, 1.0)
    pltpu.sync_copy(zcnt, cacc.at[pl.ds(s * RPT, RPT)])
    plsc.subcore_barrier()

    def run(dst_hbm, cnt_hbm):
        for q in range(NQ):
            base = s * NCH + q * QCH
            pltpu.sync_copy(dst_hbm.at[pl.ds(base, QCH)], dst_v)

            def body(j, carry):
                pltpu.sync_copy(ones_v, cacc.at[dst_v.at[j]], add=True)
                return carry

            lax.fori_loop(0, QCH, body, 0)
        plsc.subcore_barrier()
        pltpu.sync_copy(cacc.at[pl.ds(s * RPT, RPT)],
                        cnt_hbm.at[pl.ds(s * RPT, RPT)])

    @pl.when(c == 0)
    def _():
        run(dstA, cnt_a)

    @pl.when(c == 1)
    def _():
        run(dstB, cnt_b)


_cnt = functools.partial(
    pl.kernel,
    out_type=[
        jax.ShapeDtypeStruct((NP,), jnp.float32),
        jax.ShapeDtypeStruct((NP,), jnp.float32),
    ],
    mesh=_MESH,
    scratch_types=[
        pltpu.VMEM_SHARED((NP,), jnp.float32),
        pltpu.VMEM((QCH, C), jnp.int32),
        pltpu.VMEM((RPT,), jnp.float32),
        pltpu.VMEM((C,), jnp.float32),
    ],
)(_cnt_body)


def _agg2_body(ya, yb, srcA, dstA, srcB, dstB,
               out_a, out_b,
               acc, src_v, dst_v, g0, g1, g2, g3,
               sem0, sem1, sem2, sem3):
    c = lax.axis_index("c")
    s = lax.axis_index("s")
    _fill2d(g0, C, Z, 0.0)
    for k in range(RPT // C):
        pltpu.sync_copy(g0, acc.at[pl.ds(s * RPT + k * C, C)])
    plsc.subcore_barrier()
    gs = [g0, g1, g2, g3]
    sems = [sem0, sem1, sem2, sem3]

    def run(x_hbm, src_hbm, dst_hbm, out_hbm):
        _seg_loop4(s, x_hbm, src_hbm, dst_hbm, src_v, dst_v, gs, acc, sems)
        plsc.subcore_barrier()
        pltpu.sync_copy(acc.at[pl.ds(s * RPT, RPT)],
                        out_hbm.at[pl.ds(s * RPT, RPT)])

    @pl.when(c == 0)
    def _():
        run(ya, srcA, dstA, out_a)

    @pl.when(c == 1)
    def _():
        run(yb, srcB, dstB, out_b)


_agg2 = functools.partial(
    pl.kernel,
    out_type=[
        jax.ShapeDtypeStruct((NP, Z), jnp.float32),
        jax.ShapeDtypeStruct((NP, Z), jnp.float32),
    ],
    mesh=_MESH,
    compiler_params=pltpu.CompilerParams(use_tc_tiling_on_sc=False),
    scratch_types=[
        pltpu.VMEM_SHARED((NP, Z), jnp.float32),
        pltpu.VMEM((QCH, C), jnp.int32),
        pltpu.VMEM((QCH, C), jnp.int32),
        pltpu.VMEM((C, Z), jnp.float32),
        pltpu.VMEM((C, Z), jnp.float32),
        pltpu.VMEM((C, Z), jnp.float32),
        pltpu.VMEM((C, Z), jnp.float32),
        pltpu.SemaphoreType.DMA,
        pltpu.SemaphoreType.DMA,
        pltpu.SemaphoreType.DMA,
        pltpu.SemaphoreType.DMA,
    ],
)(_agg2_body)


BM = 512
BMF = 400


def _tc1_body(m_l, cnt_l, x_l, wl1A, wr1A, b1A, wsA, waA,
              m_p, cnt_p, x_p, wl1B, wr1B, b1B, wsB, waB,
              ysl_ref, yal_ref, ysp_ref, yap_ref):
    def one(m_ref, cnt_ref, x_ref, wl1, wr1, b1, ws, wa, ys_ref, ya_ref):
        r = 1.0 / jnp.maximum(cnt_ref[...], 1.0)
        m = m_ref[...] * r
        h = jnp.dot(m, wl1[...], preferred_element_type=jnp.float32)
        h += jnp.dot(x_ref[...], wr1[...], preferred_element_type=jnp.float32)
        h += b1[...]
        h = jnp.maximum(h, 0.0)
        ys_ref[...] = jnp.dot(h, ws[...], preferred_element_type=jnp.float32)
        ya_ref[...] = jnp.dot(h, wa[...], preferred_element_type=jnp.float32)
    one(m_l, cnt_l, x_l, wl1A, wr1A, b1A, wsA, waA, ysl_ref, yal_ref)
    one(m_p, cnt_p, x_p, wl1B, wr1B, b1B, wsB, waB, ysp_ref, yap_ref)


def _tc1(m_l, cnt_l, x_l, wA, m_p, cnt_p, x_p, wB):
    grid = (N // BMF,)
    full = lambda a: pl.BlockSpec(a.shape, lambda i: tuple(0 for _ in a.shape))
    rowd = pl.BlockSpec((BMF, D), lambda i: (i, 0))
    rowc = pl.BlockSpec((BMF, 1), lambda i: (i, 0))
    rowz = pl.BlockSpec((BMF, Z), lambda i: (i, 0))
    return pl.pallas_call(
        _tc1_body,
        grid=grid,
        in_specs=[rowd, rowc, rowd] + [full(w) for w in wA]
                 + [rowd, rowc, rowd] + [full(w) for w in wB],
        out_specs=[rowz, rowz, rowz, rowz],
        out_shape=[jax.ShapeDtypeStruct((N, Z), jnp.float32)] * 4,
    )(m_l, cnt_l, x_l, *wA, m_p, cnt_p, x_p, *wB)


def _fin_body(m2l_ref, cntl_ref, ysl_ref, b2l_ref,
              m2p_ref, cntp_ref, ysp_ref, b2p_ref, zl_ref, zp_ref):
    rl = 1.0 / jnp.maximum(cntl_ref[...], 1.0)
    zl_ref[...] = m2l_ref[...] * rl + ysl_ref[...] + b2l_ref[...]
    rp = 1.0 / jnp.maximum(cntp_ref[...], 1.0)
    zp_ref[...] = m2p_ref[...] * rp + ysp_ref[...] + b2p_ref[...]


def _fin(m2l, cntl, ysl, b2l, m2p, cntp, ysp, b2p):
    grid = (N // BMF,)
    rowz = pl.BlockSpec((BMF, Z), lambda i: (i, 0))
    rowc = pl.BlockSpec((BMF, 1), lambda i: (i, 0))
    bias = pl.BlockSpec((1, Z), lambda i: (0, 0))
    return pl.pallas_call(
        _fin_body,
        grid=grid,
        in_specs=[rowz, rowc, rowz, bias, rowz, rowc, rowz, bias],
        out_specs=[rowz, rowz],
        out_shape=[
            jax.ShapeDtypeStruct((N, Z), jnp.float32),
            jax.ShapeDtypeStruct((N, Z), jnp.float32),
        ],
    )(m2l, cntl, ysl, b2l, m2p, cntp, ysp, b2p)


def kernel(x_paper, x_label, edge_index_p2l, edge_index_l2p,
           Wl1_p2l, Wr1_p2l, b1_p2l, Wl1_l2p, Wr1_l2p, b1_l2p,
           Wl2_p2l, Wr2_p2l, b2_p2l, Wl2_l2p, Wr2_l2p, b2_l2p):
    xp = x_paper
    xl = x_label

    def prep(ei):
        src = ei[0].astype(jnp.int32)
        dst = ei[1].astype(jnp.int32)
        pad = EP - E
        src = jnp.concatenate([src, jnp.zeros((pad,), jnp.int32)])
        dst = jnp.concatenate([dst, jnp.full((pad,), N, jnp.int32)])
        return src.reshape(16 * NCH, C), dst.reshape(16 * NCH, C)

    srcA, dstA = prep(edge_index_p2l)
    srcB, dstB = prep(edge_index_l2p)

    cnt_lab, cnt_pap = _cnt(dstA, dstB)
    cl = cnt_lab[:, None]
    cp = cnt_pap[:, None]

    m_lab, m_pap = _agg1(xp, xl, srcA, dstA, srcB, dstB)

    ys_lab, ya_lab, ys_pap, ya_pap = _tc1(
        m_lab, cl, xl, (Wl1_p2l, Wr1_p2l, b1_p2l[None], Wr2_p2l, Wl2_l2p),
        m_pap, cp, xp, (Wl1_l2p, Wr1_l2p, b1_l2p[None], Wr2_l2p, Wl2_p2l))

    m2_lab, m2_pap = _agg2(ya_pap, ya_lab, srcA, dstA, srcB, dstB)

    zl, zp = _fin(m2_lab, cl, ys_lab, b2_p2l[None],
                  m2_pap, cp, ys_pap, b2_l2p[None])
    return (zp, zl)
